# Initial kernel scaffold; baseline (speedup 1.0000x reference)
import jax
import jax.numpy as jnp
from jax.experimental import pallas as pl


def kernel(logits, k, p):
    raise NotImplementedError("write your pallas kernel here")


# speedup vs baseline: 85.8023x; 85.8023x over previous
import jax
import jax.numpy as jnp
import numpy as np
from jax.experimental import pallas as pl
from jax.experimental.pallas import tpu as pltpu

B = 64
N = 100000
ROWS = 16

_I32_MIN = np.int32(-2147483648)
_I32_MAX = np.int32(2147483647)


def _key_from_float(x):
    b = jax.lax.bitcast_convert_type(x, jnp.int32)
    kneg = jax.lax.bitwise_xor(jnp.bitwise_not(b), _I32_MIN)
    return jnp.where(b >= 0, b, kneg)


def _float_from_key(t):
    bneg = jnp.bitwise_not(jax.lax.bitwise_xor(t, _I32_MIN))
    return jax.lax.bitcast_convert_type(jnp.where(t >= 0, t, bneg),
                                        jnp.float32)


_KEY8 = np.int32(0x41000000)


def _body(x_ref, k_ref, p_ref, o_ref):
    x = x_ref[...]
    kk = k_ref[...]
    pp = p_ref[...]

    key = _key_from_float(x)
    m = jnp.max(x, axis=1, keepdims=True)
    ex = jnp.exp(x - m)

    c0 = jnp.sum(jnp.where(key >= 0, 1.0, 0.0), axis=1, keepdims=True)
    lo0 = jnp.where(c0 >= kk.astype(jnp.float32), jnp.int32(0),
                    jnp.min(key, axis=1, keepdims=True))
    hi0 = jnp.max(key, axis=1, keepdims=True) + 1

    def s1_cond(c):
        it, lo, hi = c
        return (it < 33) & (jnp.max(hi - lo) > 1)

    def s1(c):
        it, lo, hi = c
        mid = lo + jax.lax.shift_right_logical(hi - lo, 1)
        cnt = jnp.sum(jnp.where(key >= mid, 1.0, 0.0), axis=1, keepdims=True)
        pred = cnt >= kk.astype(jnp.float32)
        return it + 1, jnp.where(pred, mid, lo), jnp.where(pred, hi, mid)

    _, thr_key, _ = jax.lax.while_loop(s1_cond, s1, (jnp.int32(0), lo0, hi0))

    idx = jax.lax.broadcasted_iota(jnp.int32, x.shape, 1)

    def _post(use_ex):
        def post(_):
            if use_ex:
                ge_thr = ex >= jnp.exp(_float_from_key(thr_key) - m)
            else:
                ge_thr = key >= thr_key
            z = jnp.sum(jnp.where(ge_thr, ex, 0.0), axis=1, keepdims=True)
            pz = pp * z

            def s2_cond(c):
                it, lo, hi = c
                return (it < 32) & (jnp.max(hi - lo) > 1)

            def s2(c):
                it, lo, hi = c
                mid = lo + jax.lax.shift_right_logical(hi - lo, 1)
                if use_ex:
                    msk = ex > jnp.exp(_float_from_key(mid) - m)
                else:
                    msk = key > mid
                f = jnp.sum(jnp.where(msk, ex, 0.0), axis=1, keepdims=True)
                pred = f < pz
                return (it + 1, jnp.where(pred, lo, mid),
                        jnp.where(pred, mid, hi))

            _, _, t_min = jax.lax.while_loop(
                s2_cond, s2, (jnp.int32(0), thr_key - 1, hi0 - 1))

            if use_ex:
                ge = ex >= jnp.exp(_float_from_key(t_min) - m)
                ex_t = jnp.min(jnp.where(ge, ex, jnp.inf), axis=1,
                               keepdims=True)
                at_t = ex == ex_t
                gt_t = ex > ex_t
            else:
                ge = key >= t_min
                t_key = jnp.min(jnp.where(ge, key, _I32_MAX), axis=1,
                                keepdims=True)
                at_t = key == t_key
                gt_t = key > t_key
                ex_t = jnp.max(jnp.where(at_t, ex, 0.0), axis=1,
                               keepdims=True)
            f_t = jnp.sum(jnp.where(gt_t, ex, 0.0), axis=1, keepdims=True)
            rhs = pz - f_t

            idxmax = jnp.max(jnp.where(at_t, idx, -1), axis=1, keepdims=True)
            need_slow = jnp.sum(jnp.where(ex_t < rhs, 1.0, 0.0)) > 0.0

            def slow(_):
                def s3(_, lh):
                    lo, hi = lh
                    mid = lo + (hi - lo) // 2
                    d = jnp.sum(jnp.where(at_t & (idx >= mid), 1.0, 0.0),
                                axis=1, keepdims=True)
                    pred = (d - 1.0) * ex_t < rhs
                    return jnp.where(pred, lo, mid), jnp.where(pred, mid, hi)

                neg1 = jnp.full_like(thr_key, -1)
                topn = jnp.full_like(thr_key, N)
                _, im = jax.lax.fori_loop(0, 17, s3, (neg1, topn))
                return im

            i_min = jax.lax.cond(need_slow, slow, lambda _: idxmax, None)
            keep = ge & (jnp.logical_not(at_t) | (idx >= i_min))
            o_ref[...] = jnp.where(keep, x, -jnp.inf)
            return jnp.int32(0)
        return post

    guard = (jnp.min(thr_key) >= _KEY8) & (jnp.max(m) < 24.0)
    jax.lax.cond(guard, _post(True), _post(False), None)


@jax.jit
def kernel(logits, k, p):
    k2 = k.reshape(B, 1)
    p2 = p.reshape(B, 1)
    grid = (B // ROWS,)
    return pl.pallas_call(
        _body,
        grid=grid,
        in_specs=[
            pl.BlockSpec((ROWS, N), lambda i: (i, 0)),
            pl.BlockSpec((ROWS, 1), lambda i: (i, 0)),
            pl.BlockSpec((ROWS, 1), lambda i: (i, 0)),
        ],
        out_specs=pl.BlockSpec((ROWS, N), lambda i: (i, 0)),
        out_shape=jax.ShapeDtypeStruct((B, N), jnp.float32),
        interpret=False,
    )(logits, k2, p2)
